# baseline (speedup 1.0000x reference)
import functools

import jax
import jax.numpy as jnp
from jax.experimental import pallas as pl
from jax.experimental.pallas import tpu as pltpu

_N = 262144
_K = 512
_D = 32
_BN = 2048
_DP = 40


def _body(ea_ref, ca_ref, ids_ref, loss_ref):
    i = pl.program_id(0)
    ea = ea_ref[...]
    ca = ca_ref[...]
    s = jax.lax.dot_general(
        ca, ea, (((1,), (0,)), ((), ())),
        precision=jax.lax.Precision.HIGHEST,
        preferred_element_type=jnp.float32)
    m = jnp.min(s, axis=0, keepdims=True)
    iota = jax.lax.broadcasted_iota(jnp.int32, s.shape, 0)
    ids = jnp.min(jnp.where(s == m, iota, _K), axis=0)
    ids_ref[...] = ids

    e = ea[:_D, :]
    part = jnp.sum(e * e) + jnp.sum(m)

    @pl.when(i == 0)
    def _():
        loss_ref[0, 0] = 0.0

    loss_ref[0, 0] += part


@jax.jit
def _cluster(embs, centers):
    c2 = jnp.sum(centers * centers, axis=1, keepdims=True)
    ca = jnp.concatenate(
        [-2.0 * centers, c2, jnp.zeros((_K, _DP - _D - 1), jnp.float32)],
        axis=1)
    eaT = jnp.concatenate(
        [embs.T, jnp.ones((1, _N), jnp.float32),
         jnp.zeros((_DP - _D - 1, _N), jnp.float32)], axis=0)
    grid = _N // _BN
    ids, loss = pl.pallas_call(
        _body,
        grid=(grid,),
        in_specs=[
            pl.BlockSpec((_DP, _BN), lambda i: (0, i)),
            pl.BlockSpec((_K, _DP), lambda i: (0, 0)),
        ],
        out_specs=[
            pl.BlockSpec((_BN,), lambda i: (i,)),
            pl.BlockSpec((1, 1), lambda i: (0, 0), memory_space=pltpu.SMEM),
        ],
        out_shape=[
            jax.ShapeDtypeStruct((_N,), jnp.int32),
            jax.ShapeDtypeStruct((1, 1), jnp.float32),
        ],
    )(eaT, ca)
    return ids, loss[0, 0]


def kernel(embs, centers):
    ids, loss = _cluster(embs, centers)
    return (centers, ids, loss)


# speedup vs baseline: 1.8246x; 1.8246x over previous
import functools

import jax
import jax.numpy as jnp
from jax.experimental import pallas as pl
from jax.experimental.pallas import tpu as pltpu

_N = 262144
_K = 512
_D = 32
_BN = 2048
_DP = 40


def _body(ea_ref, ca_ref, ids_ref, loss_ref):
    i = pl.program_id(0)
    ea = ea_ref[...]
    ca = ca_ref[...]
    s = jax.lax.dot_general(
        ca, ea, (((1,), (1,)), ((), ())),
        preferred_element_type=jnp.float32)
    m = jnp.min(s, axis=0, keepdims=True)
    iota = jax.lax.broadcasted_iota(jnp.int32, s.shape, 0)
    ids = jnp.min(jnp.where(s == m, iota, _K), axis=0)
    ids_ref[...] = ids

    part = (jnp.sum(ea * ea) - _BN) + jnp.sum(m)

    @pl.when(i == 0)
    def _():
        loss_ref[0, 0] = 0.0

    loss_ref[0, 0] += part


@jax.jit
def _cluster(embs, centers):
    c2 = jnp.sum(centers * centers, axis=1, keepdims=True)
    ca = jnp.concatenate(
        [-2.0 * centers, c2, jnp.zeros((_K, _DP - _D - 1), jnp.float32)],
        axis=1)
    ea = jnp.concatenate(
        [embs, jnp.ones((_N, 1), jnp.float32),
         jnp.zeros((_N, _DP - _D - 1), jnp.float32)], axis=1)
    grid = _N // _BN
    ids, loss = pl.pallas_call(
        _body,
        grid=(grid,),
        in_specs=[
            pl.BlockSpec((_BN, _DP), lambda i: (i, 0)),
            pl.BlockSpec((_K, _DP), lambda i: (0, 0)),
        ],
        out_specs=[
            pl.BlockSpec((_BN,), lambda i: (i,)),
            pl.BlockSpec((1, 1), lambda i: (0, 0), memory_space=pltpu.SMEM),
        ],
        out_shape=[
            jax.ShapeDtypeStruct((_N,), jnp.int32),
            jax.ShapeDtypeStruct((1, 1), jnp.float32),
        ],
    )(ea, ca)
    return ids, loss[0, 0]


def kernel(embs, centers):
    ids, loss = _cluster(embs, centers)
    return (centers, ids, loss)


# speedup vs baseline: 2.0611x; 1.1296x over previous
import functools

import jax
import jax.numpy as jnp
from jax.experimental import pallas as pl
from jax.experimental.pallas import tpu as pltpu

_N = 262144
_K = 512
_D = 32
_BN = 4096


def _body(e_ref, cm2_ref, c2_ref, ids_ref, loss_ref):
    i = pl.program_id(0)
    e = e_ref[...]
    cm2 = cm2_ref[...]
    s = jax.lax.dot_general(
        cm2, e, (((1,), (1,)), ((), ())),
        preferred_element_type=jnp.float32)
    s = s + c2_ref[...]
    m = jnp.min(s, axis=0, keepdims=True)
    iota = jax.lax.broadcasted_iota(jnp.int32, s.shape, 0)
    ids = jnp.min(jnp.where(s == m, iota, _K), axis=0)
    ids_ref[...] = ids

    part = jnp.sum(e * e) + jnp.sum(m)

    @pl.when(i == 0)
    def _():
        loss_ref[0, 0] = 0.0

    loss_ref[0, 0] += part


@jax.jit
def _cluster(embs, centers):
    cm2 = -2.0 * centers
    c2 = jnp.sum(centers * centers, axis=1, keepdims=True)
    grid = _N // _BN
    ids, loss = pl.pallas_call(
        _body,
        grid=(grid,),
        in_specs=[
            pl.BlockSpec((_BN, _D), lambda i: (i, 0)),
            pl.BlockSpec((_K, _D), lambda i: (0, 0)),
            pl.BlockSpec((_K, 1), lambda i: (0, 0)),
        ],
        out_specs=[
            pl.BlockSpec((_BN,), lambda i: (i,)),
            pl.BlockSpec((1, 1), lambda i: (0, 0), memory_space=pltpu.SMEM),
        ],
        out_shape=[
            jax.ShapeDtypeStruct((_N,), jnp.int32),
            jax.ShapeDtypeStruct((1, 1), jnp.float32),
        ],
    )(embs, cm2, c2)
    return ids, loss[0, 0]


def kernel(embs, centers):
    ids, loss = _cluster(embs, centers)
    return (centers, ids, loss)


# speedup vs baseline: 2.0955x; 1.0167x over previous
import functools

import jax
import jax.numpy as jnp
from jax.experimental import pallas as pl
from jax.experimental.pallas import tpu as pltpu

_N = 262144
_K = 512
_D = 32
_BN = 8192


def _body(e_ref, cm2_ref, c2_ref, ids_ref, loss_ref):
    i = pl.program_id(0)
    e = e_ref[...]
    cm2 = cm2_ref[...]
    s = jax.lax.dot_general(
        cm2, e, (((1,), (1,)), ((), ())),
        preferred_element_type=jnp.float32)
    s = s + c2_ref[...]
    m = jnp.min(s, axis=0, keepdims=True)
    iota = jax.lax.broadcasted_iota(jnp.int32, s.shape, 0)
    ids = jnp.min(jnp.where(s == m, iota, _K), axis=0)
    ids_ref[...] = ids

    part = jnp.sum(e * e) + jnp.sum(m)

    @pl.when(i == 0)
    def _():
        loss_ref[0, 0] = 0.0

    loss_ref[0, 0] += part


@jax.jit
def _cluster(embs, centers):
    cm2 = -2.0 * centers
    c2 = jnp.sum(centers * centers, axis=1, keepdims=True)
    grid = _N // _BN
    ids, loss = pl.pallas_call(
        _body,
        grid=(grid,),
        in_specs=[
            pl.BlockSpec((_BN, _D), lambda i: (i, 0)),
            pl.BlockSpec((_K, _D), lambda i: (0, 0)),
            pl.BlockSpec((_K, 1), lambda i: (0, 0)),
        ],
        out_specs=[
            pl.BlockSpec((_BN,), lambda i: (i,)),
            pl.BlockSpec((1, 1), lambda i: (0, 0), memory_space=pltpu.SMEM),
        ],
        out_shape=[
            jax.ShapeDtypeStruct((_N,), jnp.int32),
            jax.ShapeDtypeStruct((1, 1), jnp.float32),
        ],
    )(embs, cm2, c2)
    return ids, loss[0, 0]


def kernel(embs, centers):
    ids, loss = _cluster(embs, centers)
    return (centers, ids, loss)


# speedup vs baseline: 4.0494x; 1.9325x over previous
import functools

import jax
import jax.numpy as jnp
from jax.experimental import pallas as pl
from jax.experimental.pallas import tpu as pltpu

_N = 262144
_K = 512
_D = 32
_BN = 8192


def _body(e_ref, cm2_ref, c2_ref, ids_ref, loss_ref):
    i = pl.program_id(0)
    e = e_ref[...]
    cm2 = cm2_ref[...]
    s = jax.lax.dot_general(
        cm2, e, (((1,), (0,)), ((), ())),
        preferred_element_type=jnp.float32)
    s = s + c2_ref[...]
    m = jnp.min(s, axis=0, keepdims=True)
    ids_ref[...] = jnp.argmin(s, axis=0).astype(jnp.int32)

    part = jnp.sum(e * e) + jnp.sum(m)

    @pl.when(i == 0)
    def _():
        loss_ref[0, 0] = 0.0

    loss_ref[0, 0] += part


@jax.jit
def _cluster(embs, centers):
    cm2 = -2.0 * centers
    c2 = jnp.sum(centers * centers, axis=1, keepdims=True)
    eT = embs.T
    grid = _N // _BN
    ids, loss = pl.pallas_call(
        _body,
        grid=(grid,),
        in_specs=[
            pl.BlockSpec((_D, _BN), lambda i: (0, i)),
            pl.BlockSpec((_K, _D), lambda i: (0, 0)),
            pl.BlockSpec((_K, 1), lambda i: (0, 0)),
        ],
        out_specs=[
            pl.BlockSpec((_BN,), lambda i: (i,)),
            pl.BlockSpec((1, 1), lambda i: (0, 0), memory_space=pltpu.SMEM),
        ],
        out_shape=[
            jax.ShapeDtypeStruct((_N,), jnp.int32),
            jax.ShapeDtypeStruct((1, 1), jnp.float32),
        ],
    )(eT, cm2, c2)
    return ids, loss[0, 0]


def kernel(embs, centers):
    ids, loss = _cluster(embs, centers)
    return (centers, ids, loss)


# speedup vs baseline: 4.1289x; 1.0196x over previous
import functools

import jax
import jax.numpy as jnp
from jax.experimental import pallas as pl
from jax.experimental.pallas import tpu as pltpu

_N = 262144
_K = 512
_D = 32
_BN = 16384


def _body(e_ref, cm2_ref, c2_ref, ids_ref, loss_ref):
    i = pl.program_id(0)
    e = e_ref[...]
    cm2 = cm2_ref[...]
    s = jax.lax.dot_general(
        cm2, e, (((1,), (0,)), ((), ())),
        preferred_element_type=jnp.float32)
    s = s + c2_ref[...]
    m = jnp.min(s, axis=0, keepdims=True)
    ids_ref[...] = jnp.argmin(s, axis=0).astype(jnp.int32)

    part = jnp.sum(e * e) + jnp.sum(m)

    @pl.when(i == 0)
    def _():
        loss_ref[0, 0] = 0.0

    loss_ref[0, 0] += part


@jax.jit
def _cluster(embs, centers):
    cm2 = -2.0 * centers
    c2 = jnp.sum(centers * centers, axis=1, keepdims=True)
    eT = embs.T
    grid = _N // _BN
    ids, loss = pl.pallas_call(
        _body,
        grid=(grid,),
        in_specs=[
            pl.BlockSpec((_D, _BN), lambda i: (0, i)),
            pl.BlockSpec((_K, _D), lambda i: (0, 0)),
            pl.BlockSpec((_K, 1), lambda i: (0, 0)),
        ],
        out_specs=[
            pl.BlockSpec((_BN,), lambda i: (i,)),
            pl.BlockSpec((1, 1), lambda i: (0, 0), memory_space=pltpu.SMEM),
        ],
        out_shape=[
            jax.ShapeDtypeStruct((_N,), jnp.int32),
            jax.ShapeDtypeStruct((1, 1), jnp.float32),
        ],
    )(eT, cm2, c2)
    return ids, loss[0, 0]


def kernel(embs, centers):
    ids, loss = _cluster(embs, centers)
    return (centers, ids, loss)


# speedup vs baseline: 5.3183x; 1.2881x over previous
import functools

import jax
import jax.numpy as jnp
from jax.experimental import pallas as pl
from jax.experimental.pallas import tpu as pltpu

_N = 262144
_K = 512
_D = 32
_BN = 16384


def _body(e_ref, cm2_ref, c2_ref, ids_ref, loss_ref):
    i = pl.program_id(0)
    e = e_ref[...]
    cm2 = cm2_ref[...]
    s = jax.lax.dot_general(
        cm2, e, (((1,), (0,)), ((), ())),
        preferred_element_type=jnp.float32)
    s = s + c2_ref[...]
    iota8 = jax.lax.broadcasted_iota(jnp.int32, (8, _BN), 0)
    runv = s[0:8, :]
    runi = iota8
    for r in range(1, _K // 8):
        v = s[8 * r:8 * r + 8, :]
        lt = v < runv
        runv = jnp.where(lt, v, runv)
        runi = jnp.where(lt, iota8 + 8 * r, runi)
    m8 = jnp.min(runv, axis=0, keepdims=True)
    sel = jnp.where(runv == m8, runi, _K)
    ids_ref[...] = jnp.min(sel, axis=0)

    part = jnp.sum(e * e) + jnp.sum(m8)

    @pl.when(i == 0)
    def _():
        loss_ref[0, 0] = 0.0

    loss_ref[0, 0] += part


@jax.jit
def _cluster(embs, centers):
    cm2 = -2.0 * centers
    c2 = jnp.sum(centers * centers, axis=1, keepdims=True)
    eT = embs.T
    grid = _N // _BN
    ids, loss = pl.pallas_call(
        _body,
        grid=(grid,),
        in_specs=[
            pl.BlockSpec((_D, _BN), lambda i: (0, i)),
            pl.BlockSpec((_K, _D), lambda i: (0, 0)),
            pl.BlockSpec((_K, 1), lambda i: (0, 0)),
        ],
        out_specs=[
            pl.BlockSpec((_BN,), lambda i: (i,)),
            pl.BlockSpec((1, 1), lambda i: (0, 0), memory_space=pltpu.SMEM),
        ],
        out_shape=[
            jax.ShapeDtypeStruct((_N,), jnp.int32),
            jax.ShapeDtypeStruct((1, 1), jnp.float32),
        ],
    )(eT, cm2, c2)
    return ids, loss[0, 0]


def kernel(embs, centers):
    ids, loss = _cluster(embs, centers)
    return (centers, ids, loss)
